# baseline (speedup 1.0000x reference)
import jax
import jax.numpy as jnp
from jax import lax
from jax.experimental import pallas as pl
from jax.experimental.pallas import tpu as pltpu
from jax.experimental.pallas import tpu_sc as plsc

_MARK_AX = 0
_OP_SHL = 1
_OP_SHR = 2
_ALU_LO = 3
_ALU_HI = 19
_AX_CARRY_LO = 35
_OUTPUT_LO = 51
_OUTPUT_HI = 67

_NC = 2
_NS = 16
_L = 16
_NW = _NC * _NS

_N_ROWS = 8 * 2048
_F = 96
_ROWS_PER_W = _N_ROWS // _NW
_GROUPS = _ROWS_PER_W // _L
_WORDS_PER_W = _ROWS_PER_W * _F


def _sc_body(x_hbm, out_hbm, buf):
    c = lax.axis_index("c")
    s = lax.axis_index("s")
    wid = s * _NC + c
    base = wid * _WORDS_PER_W

    pltpu.sync_copy(x_hbm.at[pl.ds(base, _WORDS_PER_W)], buf)
    lane_iota = lax.iota(jnp.int32, _L)

    def group(g, carry):
        rows_f = lane_iota * _F + g * (_L * _F)

        def gcol(col):
            return plsc.load_gather(buf, [rows_f + col])

        def wargmax(lo):
            best = gcol(lo)
            besti = jnp.zeros((_L,), jnp.int32)
            for j in range(1, 16):
                v = gcol(lo + j)
                m = v > best
                best = jnp.where(m, v, best)
                besti = jnp.where(m, j, besti)
            return besti

        val_lo = wargmax(_ALU_LO)
        val_hi = wargmax(_ALU_HI)
        shift_amt = jnp.minimum(wargmax(_AX_CARRY_LO), 31)

        active = (gcol(_MARK_AX) >= 0.5) & (
            (gcol(_OP_SHL) > 0.5) | (gcol(_OP_SHR) > 0.5))
        is_shl = gcol(_OP_SHL) > 0.5

        value = val_lo + (val_hi << 4)
        shl_res = (value << shift_amt) & 255
        shr_res = lax.shift_right_logical(value, shift_amt)
        result = jnp.where(is_shl, shl_res, shr_res)

        two = jnp.full((_L,), 2.0, jnp.float32)
        plsc.addupdate_scatter(
            buf, [rows_f + ((result & 15) + _OUTPUT_LO)], two, mask=active)
        plsc.addupdate_scatter(
            buf, [rows_f + ((result >> 4) + _OUTPUT_HI)], two, mask=active)
        return carry

    lax.fori_loop(0, _GROUPS, group, 0)

    pltpu.sync_copy(buf, out_hbm.at[pl.ds(base, _WORDS_PER_W)])


def kernel(x_bd):
    b, sq, f = x_bd.shape
    x1 = x_bd.reshape(b * sq * f)
    mesh = plsc.VectorSubcoreMesh(
        core_axis_name="c", subcore_axis_name="s",
        num_cores=_NC, num_subcores=_NS)
    run = pl.kernel(
        _sc_body,
        out_type=jax.ShapeDtypeStruct((b * sq * f,), x_bd.dtype),
        mesh=mesh,
        scratch_types=[pltpu.VMEM((_WORDS_PER_W,), jnp.float32)],
        compiler_params=pltpu.CompilerParams(needs_layout_passes=False),
    )
    out = run(x1)
    return out.reshape(b, sq, f)


# speedup vs baseline: 1.5324x; 1.5324x over previous
import jax
import jax.numpy as jnp
from jax import lax
from jax.experimental import pallas as pl
from jax.experimental.pallas import tpu as pltpu
from jax.experimental.pallas import tpu_sc as plsc

_MARK_AX = 0
_OP_SHL = 1
_OP_SHR = 2
_ALU_LO = 3
_ALU_HI = 19
_AX_CARRY_LO = 35
_OUTPUT_LO = 51
_OUTPUT_HI = 67

_NC = 2
_NS = 16
_L = 16
_NW = _NC * _NS

_B = 8
_S = 2048
_F = 96
_ROWS_PER_W = _B * _S // _NW
_GROUPS = _ROWS_PER_W // _L
_W_PER_B = _S // _ROWS_PER_W


def _sc_body(x_hbm, out_hbm, buf):
    c = lax.axis_index("c")
    s = lax.axis_index("s")
    wid = s * _NC + c
    bi = wid // _W_PER_B
    r0 = (wid % _W_PER_B) * _ROWS_PER_W

    pltpu.sync_copy(x_hbm.at[bi, pl.ds(r0, _ROWS_PER_W)], buf)
    lane_iota = lax.iota(jnp.int32, _L)

    def group(g, carry):
        rows = lane_iota + g * _L

        def gcol(col):
            return plsc.load_gather(buf, [rows, jnp.full((_L,), col, jnp.int32)])

        def wargmax(lo):
            best = gcol(lo)
            besti = jnp.zeros((_L,), jnp.int32)
            for j in range(1, 16):
                v = gcol(lo + j)
                m = v > best
                best = jnp.where(m, v, best)
                besti = jnp.where(m, j, besti)
            return besti

        val_lo = wargmax(_ALU_LO)
        val_hi = wargmax(_ALU_HI)
        shift_amt = jnp.minimum(wargmax(_AX_CARRY_LO), 31)

        active = (gcol(_MARK_AX) >= 0.5) & (
            (gcol(_OP_SHL) > 0.5) | (gcol(_OP_SHR) > 0.5))
        is_shl = gcol(_OP_SHL) > 0.5

        value = val_lo + (val_hi << 4)
        shl_res = (value << shift_amt) & 255
        shr_res = lax.shift_right_logical(value, shift_amt)
        result = jnp.where(is_shl, shl_res, shr_res)

        two = jnp.full((_L,), 2.0, jnp.float32)
        plsc.addupdate_scatter(
            buf, [rows, (result & 15) + _OUTPUT_LO], two, mask=active)
        plsc.addupdate_scatter(
            buf, [rows, (result >> 4) + _OUTPUT_HI], two, mask=active)
        return carry

    lax.fori_loop(0, _GROUPS, group, 0)

    pltpu.sync_copy(buf, out_hbm.at[bi, pl.ds(r0, _ROWS_PER_W)])


def kernel(x_bd):
    b, sq, f = x_bd.shape
    mesh = plsc.VectorSubcoreMesh(
        core_axis_name="c", subcore_axis_name="s",
        num_cores=_NC, num_subcores=_NS)
    run = pl.kernel(
        _sc_body,
        out_type=jax.ShapeDtypeStruct((b, sq, f), x_bd.dtype),
        mesh=mesh,
        scratch_types=[pltpu.VMEM((_ROWS_PER_W, f), jnp.float32)],
        compiler_params=pltpu.CompilerParams(needs_layout_passes=False),
    )
    return run(x_bd)
